# initial kernel scaffold (unmeasured)
import jax
import jax.numpy as jnp
from jax import lax
from jax.experimental import pallas as pl
from jax.experimental.pallas import tpu as pltpu


def kernel(
    x,
):
    def body(*refs):
        pass

    out_shape = jax.ShapeDtypeStruct(..., jnp.float32)
    return pl.pallas_call(body, out_shape=out_shape)(...)


# baseline (device time: 121950 ns/iter reference)
import functools

import jax
import jax.numpy as jnp
from jax import lax
from jax.experimental import pallas as pl
from jax.experimental.pallas import tpu as pltpu

N_LISTS = 32
K = 32
Z = 4
ROW_BLOCK = 128


def _bitonic_sort32_desc(Y):
    n = N_LISTS
    k = 2
    while k <= n:
        d = k // 2
        while d >= 1:
            for i in range(n):
                l = i ^ d
                if l > i:
                    a, b = Y[i], Y[l]
                    if (i & k) == 0:
                        Y[i], Y[l] = jnp.maximum(a, b), jnp.minimum(a, b)
                    else:
                        Y[i], Y[l] = jnp.minimum(a, b), jnp.maximum(a, b)
            d //= 2
        k *= 2
    return Y


def _bitonic_merge32_desc(C):
    for d in (16, 8, 4, 2, 1):
        for i in range(N_LISTS):
            l = i + d
            if (i & d) == 0 and l < N_LISTS:
                a, b = C[i], C[l]
                C[i], C[l] = jnp.maximum(a, b), jnp.minimum(a, b)
    return C


def _local_topk32(x):
    _, c = x.shape
    L = c // N_LISTS
    Y = [x[:, L * j : L * (j + 1)] for j in range(N_LISTS)]
    Y = _bitonic_sort32_desc(Y)
    while L > 1:
        h = L // 2
        C = [
            jnp.maximum(Y[j][:, :h], Y[N_LISTS - 1 - j][:, h:])
            for j in range(N_LISTS)
        ]
        Y = _bitonic_merge32_desc(C)
        L = h
    return jnp.concatenate(Y, axis=1)


def _merge_extract(C, k=K):
    r, m = C.shape
    iota = lax.broadcasted_iota(jnp.int32, (r, m), 1)
    cols = []
    for _ in range(k):
        mx = jnp.max(C, axis=1, keepdims=True)
        t = jnp.where(C == mx, iota, m)
        jm = jnp.min(t, axis=1, keepdims=True)
        C = jnp.where(iota == jm, -jnp.inf, C)
        cols.append(mx)
    return jnp.concatenate(cols, axis=1)


def _local_body(x_ref, o_ref):
    o_ref[...] = _local_topk32(x_ref[...])


def _comm_body(loc_ref, out_ref, comm_ref, send_sems, recv_sems):
    my_x = lax.axis_index("x")
    my_y = lax.axis_index("y")
    my_z = lax.axis_index("z")

    barrier = pltpu.get_barrier_semaphore()
    for dz in (1, 2, 3):
        pl.semaphore_signal(
            barrier,
            inc=1,
            device_id=(my_x, my_y, (my_z + dz) % Z),
            device_id_type=pl.DeviceIdType.MESH,
        )
    pl.semaphore_wait(barrier, Z - 1)

    sends = []
    for dz in (1, 2, 3):
        rdma = pltpu.make_async_remote_copy(
            src_ref=loc_ref,
            dst_ref=comm_ref.at[dz - 1],
            send_sem=send_sems.at[dz - 1],
            recv_sem=recv_sems.at[dz - 1],
            device_id=(my_x, my_y, (my_z + dz) % Z),
            device_id_type=pl.DeviceIdType.MESH,
        )
        rdma.start()
        sends.append(rdma)
    for rdma in sends:
        rdma.wait_recv()
    for rdma in sends:
        rdma.wait_send()

    cands = jnp.concatenate(
        [loc_ref[...], comm_ref[0], comm_ref[1], comm_ref[2]], axis=1
    )
    out_ref[...] = _merge_extract(cands)

    @functools.partial(pl.run_scoped, sem=pltpu.SemaphoreType.REGULAR)
    def _(sem):
        for dz in (1, 2, 3):
            pl.semaphore_signal(
                sem,
                inc=1,
                device_id=(my_x, my_y, (my_z + dz) % Z),
                device_id_type=pl.DeviceIdType.MESH,
            )
        pl.semaphore_wait(sem, Z - 1)


def kernel(x):
    m, n = x.shape

    local = pl.pallas_call(
        _local_body,
        grid=(m // ROW_BLOCK,),
        in_specs=[
            pl.BlockSpec((ROW_BLOCK, n), lambda i: (i, 0)),
        ],
        out_specs=pl.BlockSpec((ROW_BLOCK, K), lambda i: (i, 0)),
        out_shape=jax.ShapeDtypeStruct((m, K), jnp.float32),
    )(x)

    return pl.pallas_call(
        _comm_body,
        out_shape=jax.ShapeDtypeStruct((m, K), jnp.float32),
        in_specs=[pl.BlockSpec(memory_space=pltpu.VMEM)],
        out_specs=pl.BlockSpec(memory_space=pltpu.VMEM),
        scratch_shapes=[
            pltpu.VMEM((Z - 1, m, K), jnp.float32),
            pltpu.SemaphoreType.DMA((Z - 1,)),
            pltpu.SemaphoreType.DMA((Z - 1,)),
        ],
        compiler_params=pltpu.CompilerParams(collective_id=0),
    )(local)


# device time: 40012 ns/iter; 3.0478x vs baseline; 3.0478x over previous
import functools

import jax
import jax.numpy as jnp
from jax import lax
from jax.experimental import pallas as pl
from jax.experimental.pallas import tpu as pltpu

N_LISTS = 32
K = 32
Z = 4
P = 8
RB = 128


def _ring_index(x_idx, y_idx):
    return jnp.where(x_idx == 0, y_idx, 2 * Z - 1 - y_idx)


def _ring_coords(p):
    px = p // Z
    py = jnp.where(px == 0, p, 2 * Z - 1 - p)
    return px, py


def _bitonic_sort32_desc(Y):
    n = N_LISTS
    k = 2
    while k <= n:
        d = k // 2
        while d >= 1:
            for i in range(n):
                l = i ^ d
                if l > i:
                    a, b = Y[i], Y[l]
                    if (i & k) == 0:
                        Y[i], Y[l] = jnp.maximum(a, b), jnp.minimum(a, b)
                    else:
                        Y[i], Y[l] = jnp.minimum(a, b), jnp.maximum(a, b)
            d //= 2
        k *= 2
    return Y


def _bitonic_merge32_desc(C):
    for d in (16, 8, 4, 2, 1):
        for i in range(N_LISTS):
            l = i + d
            if (i & d) == 0 and l < N_LISTS:
                a, b = C[i], C[l]
                C[i], C[l] = jnp.maximum(a, b), jnp.minimum(a, b)
    return C


def _local_topk32(x):
    _, c = x.shape
    L = c // N_LISTS
    Y = [x[:, L * j : L * (j + 1)] for j in range(N_LISTS)]
    Y = _bitonic_sort32_desc(Y)
    while L > 1:
        h = L // 2
        C = [
            jnp.maximum(Y[j][:, :h], Y[N_LISTS - 1 - j][:, h:])
            for j in range(N_LISTS)
        ]
        Y = _bitonic_merge32_desc(C)
        L = h
    return jnp.concatenate(Y, axis=1)


def _merge_extract(C, k=K):
    r, m = C.shape
    iota = lax.broadcasted_iota(jnp.int32, (r, m), 1)
    cols = []
    for _ in range(k):
        mx = jnp.max(C, axis=1, keepdims=True)
        t = jnp.where(C == mx, iota, m)
        jm = jnp.min(t, axis=1, keepdims=True)
        C = jnp.where(iota == jm, -jnp.inf, C)
        cols.append(mx)
    return jnp.concatenate(cols, axis=1)


def _local_body(x_hbm_ref, o_ref, xb_ref, copy_sem):
    r = _ring_index(lax.axis_index("x"), lax.axis_index("y"))
    cp = pltpu.make_async_copy(
        x_hbm_ref.at[pl.ds(r * RB, RB), :], xb_ref, copy_sem
    )
    cp.start()
    cp.wait()
    o_ref[...] = _local_topk32(xb_ref[...])


def _comm_body(
    loc_ref, out_ref, comm1, mine_ref, comm2, s1, r1, s2, r2
):
    my_x = lax.axis_index("x")
    my_y = lax.axis_index("y")
    my_z = lax.axis_index("z")
    r = _ring_index(my_x, my_y)

    def z_peer(dz):
        return (my_x, my_y, (my_z + dz) % Z)

    def xy_peer(d):
        px, py = _ring_coords((r + d) % P)
        return (px, py, my_z)

    peers = [z_peer(dz) for dz in (1, 2, 3)] + [xy_peer(d) for d in range(1, P)]

    barrier = pltpu.get_barrier_semaphore()
    for dev in peers:
        pl.semaphore_signal(
            barrier, inc=1, device_id=dev, device_id_type=pl.DeviceIdType.MESH
        )
    pl.semaphore_wait(barrier, len(peers))

    sends1 = []
    for dz in (1, 2, 3):
        rdma = pltpu.make_async_remote_copy(
            src_ref=loc_ref,
            dst_ref=comm1.at[dz - 1],
            send_sem=s1.at[dz - 1],
            recv_sem=r1.at[dz - 1],
            device_id=z_peer(dz),
            device_id_type=pl.DeviceIdType.MESH,
        )
        rdma.start()
        sends1.append(rdma)
    for rdma in sends1:
        rdma.wait_recv()

    cands = jnp.concatenate(
        [loc_ref[...], comm1[0], comm1[1], comm1[2]], axis=1
    )
    merged = _merge_extract(cands)
    mine_ref[...] = merged
    out_ref[pl.ds(r * RB, RB), :] = merged

    sends2 = []
    for d in range(1, P):
        rdma = pltpu.make_async_remote_copy(
            src_ref=mine_ref,
            dst_ref=comm2.at[d - 1],
            send_sem=s2.at[d - 1],
            recv_sem=r2.at[d - 1],
            device_id=xy_peer(d),
            device_id_type=pl.DeviceIdType.MESH,
        )
        rdma.start()
        sends2.append(rdma)
    for d in range(1, P):
        sends2[d - 1].wait_recv()
        origin = (r - d + P) % P
        out_ref[pl.ds(origin * RB, RB), :] = comm2[d - 1]

    for rdma in sends1:
        rdma.wait_send()
    for rdma in sends2:
        rdma.wait_send()

    @functools.partial(pl.run_scoped, sem=pltpu.SemaphoreType.REGULAR)
    def _(sem):
        for dev in peers:
            pl.semaphore_signal(
                sem, inc=1, device_id=dev, device_id_type=pl.DeviceIdType.MESH
            )
        pl.semaphore_wait(sem, len(peers))


def kernel(x):
    m, n = x.shape

    local = pl.pallas_call(
        _local_body,
        in_specs=[pl.BlockSpec(memory_space=pl.ANY)],
        out_specs=pl.BlockSpec(memory_space=pltpu.VMEM),
        out_shape=jax.ShapeDtypeStruct((RB, K), jnp.float32),
        scratch_shapes=[
            pltpu.VMEM((RB, n), jnp.float32),
            pltpu.SemaphoreType.DMA,
        ],
    )(x)

    return pl.pallas_call(
        _comm_body,
        out_shape=jax.ShapeDtypeStruct((m, K), jnp.float32),
        in_specs=[pl.BlockSpec(memory_space=pltpu.VMEM)],
        out_specs=pl.BlockSpec(memory_space=pltpu.VMEM),
        scratch_shapes=[
            pltpu.VMEM((Z - 1, RB, K), jnp.float32),
            pltpu.VMEM((RB, K), jnp.float32),
            pltpu.VMEM((P - 1, RB, K), jnp.float32),
            pltpu.SemaphoreType.DMA((Z - 1,)),
            pltpu.SemaphoreType.DMA((Z - 1,)),
            pltpu.SemaphoreType.DMA((P - 1,)),
            pltpu.SemaphoreType.DMA((P - 1,)),
        ],
        compiler_params=pltpu.CompilerParams(collective_id=0),
    )(local)


# device time: 34284 ns/iter; 3.5571x vs baseline; 1.1671x over previous
import functools

import jax
import jax.numpy as jnp
from jax import lax
from jax.experimental import pallas as pl
from jax.experimental.pallas import tpu as pltpu

N_LISTS = 32
K = 32
Z = 4
P = 8
RB = 128


def _ring_index(x_idx, y_idx):
    return jnp.where(x_idx == 0, y_idx, 2 * Z - 1 - y_idx)


def _ring_coords(p):
    px = p // Z
    py = jnp.where(px == 0, p, 2 * Z - 1 - p)
    return px, py


def _bitonic_sort32_desc(Y):
    n = N_LISTS
    k = 2
    while k <= n:
        d = k // 2
        while d >= 1:
            for i in range(n):
                l = i ^ d
                if l > i:
                    a, b = Y[i], Y[l]
                    if (i & k) == 0:
                        Y[i], Y[l] = jnp.maximum(a, b), jnp.minimum(a, b)
                    else:
                        Y[i], Y[l] = jnp.minimum(a, b), jnp.maximum(a, b)
            d //= 2
        k *= 2
    return Y


def _bitonic_merge32_desc(C):
    for d in (16, 8, 4, 2, 1):
        for i in range(N_LISTS):
            l = i + d
            if (i & d) == 0 and l < N_LISTS:
                a, b = C[i], C[l]
                C[i], C[l] = jnp.maximum(a, b), jnp.minimum(a, b)
    return C


def _local_topk32_list(x):
    _, c = x.shape
    L = c // N_LISTS
    Y = [x[:, L * j : L * (j + 1)] for j in range(N_LISTS)]
    Y = _bitonic_sort32_desc(Y)
    while L > 1:
        h = L // 2
        C = [
            jnp.maximum(Y[j][:, :h], Y[N_LISTS - 1 - j][:, h:])
            for j in range(N_LISTS)
        ]
        Y = _bitonic_merge32_desc(C)
        L = h
    return Y


def _slice_list(A):
    return [A[:, j : j + 1] for j in range(N_LISTS)]


def _merge2(Al, Bl):
    C = [jnp.maximum(Al[j], Bl[N_LISTS - 1 - j]) for j in range(N_LISTS)]
    return _bitonic_merge32_desc(C)


def _body(
    x_hbm, out_ref, xb, loc_ref, mine_ref, comm1, comm2,
    cp_sem, s1, r1, s2, r2,
):
    my_x = lax.axis_index("x")
    my_y = lax.axis_index("y")
    my_z = lax.axis_index("z")
    r = _ring_index(my_x, my_y)

    def z_peer(dz):
        return (my_x, my_y, (my_z + dz) % Z)

    def xy_peer(d):
        px, py = _ring_coords((r + d) % P)
        return (px, py, my_z)

    peers = [z_peer(dz) for dz in (1, 2, 3)] + [xy_peer(d) for d in range(1, P)]

    barrier = pltpu.get_barrier_semaphore()
    for dev in peers:
        pl.semaphore_signal(
            barrier, inc=1, device_id=dev, device_id_type=pl.DeviceIdType.MESH
        )

    cp = pltpu.make_async_copy(x_hbm.at[pl.ds(r * RB, RB), :], xb, cp_sem)
    cp.start()
    cp.wait()
    mine_list = _local_topk32_list(xb[...])
    loc_ref[...] = jnp.concatenate(mine_list, axis=1)

    pl.semaphore_wait(barrier, len(peers))

    sends1 = []
    for dz in (1, 2, 3):
        rdma = pltpu.make_async_remote_copy(
            src_ref=loc_ref,
            dst_ref=comm1.at[dz - 1],
            send_sem=s1.at[dz - 1],
            recv_sem=r1.at[dz - 1],
            device_id=z_peer(dz),
            device_id_type=pl.DeviceIdType.MESH,
        )
        rdma.start()
        sends1.append(rdma)
    for rdma in sends1:
        rdma.wait_recv()

    m01 = _merge2(mine_list, _slice_list(comm1[0]))
    m23 = _merge2(_slice_list(comm1[1]), _slice_list(comm1[2]))
    merged = jnp.concatenate(_merge2(m01, m23), axis=1)

    mine_ref[...] = merged
    out_ref[pl.ds(r * RB, RB), :] = merged

    sends2 = []
    for d in range(1, P):
        rdma = pltpu.make_async_remote_copy(
            src_ref=mine_ref,
            dst_ref=comm2.at[d - 1],
            send_sem=s2.at[d - 1],
            recv_sem=r2.at[d - 1],
            device_id=xy_peer(d),
            device_id_type=pl.DeviceIdType.MESH,
        )
        rdma.start()
        sends2.append(rdma)
    for d in range(1, P):
        sends2[d - 1].wait_recv()
        origin = (r - d + P) % P
        out_ref[pl.ds(origin * RB, RB), :] = comm2[d - 1]

    for rdma in sends1:
        rdma.wait_send()
    for rdma in sends2:
        rdma.wait_send()

    @functools.partial(pl.run_scoped, sem=pltpu.SemaphoreType.REGULAR)
    def _(sem):
        for dev in peers:
            pl.semaphore_signal(
                sem, inc=1, device_id=dev, device_id_type=pl.DeviceIdType.MESH
            )
        pl.semaphore_wait(sem, len(peers))


def kernel(x):
    m, n = x.shape

    return pl.pallas_call(
        _body,
        out_shape=jax.ShapeDtypeStruct((m, K), jnp.float32),
        in_specs=[pl.BlockSpec(memory_space=pl.ANY)],
        out_specs=pl.BlockSpec(memory_space=pltpu.VMEM),
        scratch_shapes=[
            pltpu.VMEM((RB, n), jnp.float32),
            pltpu.VMEM((RB, K), jnp.float32),
            pltpu.VMEM((RB, K), jnp.float32),
            pltpu.VMEM((Z - 1, RB, K), jnp.float32),
            pltpu.VMEM((P - 1, RB, K), jnp.float32),
            pltpu.SemaphoreType.DMA,
            pltpu.SemaphoreType.DMA((Z - 1,)),
            pltpu.SemaphoreType.DMA((Z - 1,)),
            pltpu.SemaphoreType.DMA((P - 1,)),
            pltpu.SemaphoreType.DMA((P - 1,)),
        ],
        compiler_params=pltpu.CompilerParams(collective_id=0),
    )(x)


# device time: 25278 ns/iter; 4.8244x vs baseline; 1.3563x over previous
import functools

import jax
import jax.numpy as jnp
from jax import lax
from jax.experimental import pallas as pl
from jax.experimental.pallas import tpu as pltpu

N_LISTS = 32
K = 32
Z = 4
P = 8
RB = 128


def _ring_index(x_idx, y_idx):
    return jnp.where(x_idx == 0, y_idx, 2 * Z - 1 - y_idx)


def _ring_coords(p):
    px = p // Z
    py = jnp.where(px == 0, p, 2 * Z - 1 - p)
    return px, py


def _bitonic_sort32_desc(Y):
    n = N_LISTS
    k = 2
    while k <= n:
        d = k // 2
        while d >= 1:
            for i in range(n):
                l = i ^ d
                if l > i:
                    a, b = Y[i], Y[l]
                    if (i & k) == 0:
                        Y[i], Y[l] = jnp.maximum(a, b), jnp.minimum(a, b)
                    else:
                        Y[i], Y[l] = jnp.minimum(a, b), jnp.maximum(a, b)
            d //= 2
        k *= 2
    return Y


def _bitonic_merge32_desc(C):
    for d in (16, 8, 4, 2, 1):
        for i in range(N_LISTS):
            l = i + d
            if (i & d) == 0 and l < N_LISTS:
                a, b = C[i], C[l]
                C[i], C[l] = jnp.maximum(a, b), jnp.minimum(a, b)
    return C


def _local_topk32_list(x):
    _, c = x.shape
    L = c // N_LISTS
    Y = [x[:, L * j : L * (j + 1)] for j in range(N_LISTS)]
    Y = _bitonic_sort32_desc(Y)
    while L > 1:
        h = L // 2
        C = [
            jnp.maximum(Y[j][:, :h], Y[N_LISTS - 1 - j][:, h:])
            for j in range(N_LISTS)
        ]
        Y = _bitonic_merge32_desc(C)
        L = h
    return Y


def _row_list(A):
    return [A[j : j + 1, :] for j in range(N_LISTS)]


def _merge2(Al, Bl):
    C = [jnp.maximum(Al[j], Bl[N_LISTS - 1 - j]) for j in range(N_LISTS)]
    return _bitonic_merge32_desc(C)


def _body(
    x_hbm, out_ref, xb, loc_ref, comm1, comm2, cp_sem, s1, r1, s2, r2,
):
    my_x = lax.axis_index("x")
    my_y = lax.axis_index("y")
    my_z = lax.axis_index("z")
    r = _ring_index(my_x, my_y)

    def z_peer(dz):
        return (my_x, my_y, (my_z + dz) % Z)

    def xy_peer(d):
        px, py = _ring_coords((r + d) % P)
        return (px, py, my_z)

    peers = [z_peer(dz) for dz in (1, 2, 3)] + [xy_peer(d) for d in range(1, P)]

    barrier = pltpu.get_barrier_semaphore()
    for dev in peers:
        pl.semaphore_signal(
            barrier, inc=1, device_id=dev, device_id_type=pl.DeviceIdType.MESH
        )

    cp = pltpu.make_async_copy(x_hbm.at[pl.ds(r * RB, RB), :], xb, cp_sem)
    cp.start()
    cp.wait()
    mine_cols = _local_topk32_list(xb[...])
    loc_t = jnp.transpose(jnp.concatenate(mine_cols, axis=1))
    loc_ref[...] = loc_t

    pl.semaphore_wait(barrier, len(peers))

    sends1 = []
    for dz in (1, 2, 3):
        rdma = pltpu.make_async_remote_copy(
            src_ref=loc_ref,
            dst_ref=comm1.at[dz - 1],
            send_sem=s1.at[dz - 1],
            recv_sem=r1.at[dz - 1],
            device_id=z_peer(dz),
            device_id_type=pl.DeviceIdType.MESH,
        )
        rdma.start()
        sends1.append(rdma)
    for rdma in sends1:
        rdma.wait_recv()

    m01 = _merge2(_row_list(loc_ref[...]), _row_list(comm1[0]))
    m23 = _merge2(_row_list(comm1[1]), _row_list(comm1[2]))
    merged_t = jnp.concatenate(_merge2(m01, m23), axis=0)

    comm2[pl.ds(r, 1), :, :] = merged_t[None]

    sends2 = []
    for d in range(1, P):
        rdma = pltpu.make_async_remote_copy(
            src_ref=comm2.at[r],
            dst_ref=comm2.at[r],
            send_sem=s2.at[d - 1],
            recv_sem=r2.at[d - 1],
            device_id=xy_peer(d),
            device_id_type=pl.DeviceIdType.MESH,
        )
        rdma.start()
        sends2.append(rdma)
    for rdma in sends2:
        rdma.wait_recv()

    t_all = jnp.concatenate([comm2[o] for o in range(P)], axis=1)
    out_ref[...] = jnp.transpose(t_all)

    for rdma in sends1:
        rdma.wait_send()
    for rdma in sends2:
        rdma.wait_send()

    @functools.partial(pl.run_scoped, sem=pltpu.SemaphoreType.REGULAR)
    def _(sem):
        for dev in peers:
            pl.semaphore_signal(
                sem, inc=1, device_id=dev, device_id_type=pl.DeviceIdType.MESH
            )
        pl.semaphore_wait(sem, len(peers))


def kernel(x):
    m, n = x.shape

    return pl.pallas_call(
        _body,
        out_shape=jax.ShapeDtypeStruct((m, K), jnp.float32),
        in_specs=[pl.BlockSpec(memory_space=pl.ANY)],
        out_specs=pl.BlockSpec(memory_space=pltpu.VMEM),
        scratch_shapes=[
            pltpu.VMEM((RB, n), jnp.float32),
            pltpu.VMEM((K, RB), jnp.float32),
            pltpu.VMEM((Z - 1, K, RB), jnp.float32),
            pltpu.VMEM((P, K, RB), jnp.float32),
            pltpu.SemaphoreType.DMA,
            pltpu.SemaphoreType.DMA((Z - 1,)),
            pltpu.SemaphoreType.DMA((Z - 1,)),
            pltpu.SemaphoreType.DMA((P - 1,)),
            pltpu.SemaphoreType.DMA((P - 1,)),
        ],
        compiler_params=pltpu.CompilerParams(collective_id=0),
    )(x)


# device time: 21856 ns/iter; 5.5797x vs baseline; 1.1566x over previous
import functools

import jax
import jax.numpy as jnp
from jax import lax
from jax.experimental import pallas as pl
from jax.experimental.pallas import tpu as pltpu

N_LISTS = 32
K = 32
Z = 4
P = 8
RB = 128


def _ring_index(x_idx, y_idx):
    return jnp.where(x_idx == 0, y_idx, 2 * Z - 1 - y_idx)


def _ring_coords(p):
    px = p // Z
    py = jnp.where(px == 0, p, 2 * Z - 1 - p)
    return px, py


def _bitonic_sort32_desc(Y):
    n = N_LISTS
    k = 2
    while k <= n:
        d = k // 2
        while d >= 1:
            for i in range(n):
                l = i ^ d
                if l > i:
                    a, b = Y[i], Y[l]
                    if (i & k) == 0:
                        Y[i], Y[l] = jnp.maximum(a, b), jnp.minimum(a, b)
                    else:
                        Y[i], Y[l] = jnp.minimum(a, b), jnp.maximum(a, b)
            d //= 2
        k *= 2
    return Y


def _bitonic_merge32_desc(C):
    for d in (16, 8, 4, 2, 1):
        for i in range(N_LISTS):
            l = i + d
            if (i & d) == 0 and l < N_LISTS:
                a, b = C[i], C[l]
                C[i], C[l] = jnp.maximum(a, b), jnp.minimum(a, b)
    return C


STOP_L = 16


def _local_sorted_lists_t(x):
    _, c = x.shape
    L = c // N_LISTS
    Y = [x[:, L * j : L * (j + 1)] for j in range(N_LISTS)]
    Y = _bitonic_sort32_desc(Y)
    while L > STOP_L:
        h = L // 2
        C = [
            jnp.maximum(Y[j][:, :h], Y[N_LISTS - 1 - j][:, h:])
            for j in range(N_LISTS)
        ]
        Y = _bitonic_merge32_desc(C)
        L = h
    mt = jnp.transpose(jnp.concatenate(Y, axis=1))
    lists = [
        [mt[STOP_L * j + l : STOP_L * j + l + 1, :] for j in range(N_LISTS)]
        for l in range(STOP_L)
    ]
    while len(lists) > 1:
        lists = [
            _merge2(lists[2 * i], lists[2 * i + 1])
            for i in range(len(lists) // 2)
        ]
    return lists[0]


def _row_list(A):
    return [A[j : j + 1, :] for j in range(N_LISTS)]


def _merge2(Al, Bl):
    C = [jnp.maximum(Al[j], Bl[N_LISTS - 1 - j]) for j in range(N_LISTS)]
    return _bitonic_merge32_desc(C)


def _body(
    x_hbm, out_ref, xb, loc_ref, comm1, comm2, cp_sem, s1, r1, s2, r2,
):
    my_x = lax.axis_index("x")
    my_y = lax.axis_index("y")
    my_z = lax.axis_index("z")
    r = _ring_index(my_x, my_y)

    def z_peer(dz):
        return (my_x, my_y, (my_z + dz) % Z)

    def xy_peer(d):
        px, py = _ring_coords((r + d) % P)
        return (px, py, my_z)

    peers = [z_peer(dz) for dz in (1, 2, 3)] + [xy_peer(d) for d in range(1, P)]

    barrier = pltpu.get_barrier_semaphore()
    for dev in peers:
        pl.semaphore_signal(
            barrier, inc=1, device_id=dev, device_id_type=pl.DeviceIdType.MESH
        )

    hb = RB // 2
    cps = []
    for h in range(2):
        cp = pltpu.make_async_copy(
            x_hbm.at[pl.ds(r * RB + h * hb, hb), :],
            xb.at[pl.ds(h * hb, hb), :],
            cp_sem.at[h],
        )
        cp.start()
        cps.append(cp)
    half_lists = []
    for h in range(2):
        cps[h].wait()
        half_lists.append(_local_sorted_lists_t(xb[h * hb : (h + 1) * hb, :]))
    mine_lists = [
        jnp.concatenate([half_lists[0][j], half_lists[1][j]], axis=1)
        for j in range(N_LISTS)
    ]
    loc_ref[...] = jnp.concatenate(mine_lists, axis=0)

    pl.semaphore_wait(barrier, len(peers))

    sends1 = []
    for dz in (1, 2, 3):
        rdma = pltpu.make_async_remote_copy(
            src_ref=loc_ref,
            dst_ref=comm1.at[dz - 1],
            send_sem=s1.at[dz - 1],
            recv_sem=r1.at[dz - 1],
            device_id=z_peer(dz),
            device_id_type=pl.DeviceIdType.MESH,
        )
        rdma.start()
        sends1.append(rdma)
    for rdma in sends1:
        rdma.wait_recv()

    m01 = _merge2(mine_lists, _row_list(comm1[0]))
    m23 = _merge2(_row_list(comm1[1]), _row_list(comm1[2]))
    merged_t = jnp.concatenate(_merge2(m01, m23), axis=0)

    comm2[pl.ds(r, 1), :, :] = merged_t[None]

    sends2 = []
    for d in range(1, P):
        rdma = pltpu.make_async_remote_copy(
            src_ref=comm2.at[r],
            dst_ref=comm2.at[r],
            send_sem=s2.at[d - 1],
            recv_sem=r2.at[d - 1],
            device_id=xy_peer(d),
            device_id_type=pl.DeviceIdType.MESH,
        )
        rdma.start()
        sends2.append(rdma)
    for rdma in sends2:
        rdma.wait_recv()

    t_all = jnp.concatenate([comm2[o] for o in range(P)], axis=1)
    out_ref[...] = jnp.transpose(t_all)

    for rdma in sends1:
        rdma.wait_send()
    for rdma in sends2:
        rdma.wait_send()

    @functools.partial(pl.run_scoped, sem=pltpu.SemaphoreType.REGULAR)
    def _(sem):
        for dev in peers:
            pl.semaphore_signal(
                sem, inc=1, device_id=dev, device_id_type=pl.DeviceIdType.MESH
            )
        pl.semaphore_wait(sem, len(peers))


def kernel(x):
    m, n = x.shape

    return pl.pallas_call(
        _body,
        out_shape=jax.ShapeDtypeStruct((m, K), jnp.float32),
        in_specs=[pl.BlockSpec(memory_space=pl.ANY)],
        out_specs=pl.BlockSpec(memory_space=pltpu.VMEM),
        scratch_shapes=[
            pltpu.VMEM((RB, n), jnp.float32),
            pltpu.VMEM((K, RB), jnp.float32),
            pltpu.VMEM((Z - 1, K, RB), jnp.float32),
            pltpu.VMEM((P, K, RB), jnp.float32),
            pltpu.SemaphoreType.DMA((2,)),
            pltpu.SemaphoreType.DMA((Z - 1,)),
            pltpu.SemaphoreType.DMA((Z - 1,)),
            pltpu.SemaphoreType.DMA((P - 1,)),
            pltpu.SemaphoreType.DMA((P - 1,)),
        ],
        compiler_params=pltpu.CompilerParams(collective_id=0),
    )(x)


# device time: 21336 ns/iter; 5.7157x vs baseline; 1.0244x over previous
import functools

import jax
import jax.numpy as jnp
from jax import lax
from jax.experimental import pallas as pl
from jax.experimental.pallas import tpu as pltpu

N_LISTS = 32
K = 32
Z = 4
P = 8
RB = 128


def _ring_index(x_idx, y_idx):
    return jnp.where(x_idx == 0, y_idx, 2 * Z - 1 - y_idx)


def _ring_coords(p):
    px = p // Z
    py = jnp.where(px == 0, p, 2 * Z - 1 - p)
    return px, py


def _oems_pairs(n):
    pairs = []

    def oddeven_merge(lo, n2, r):
        step = r * 2
        if step < n2:
            oddeven_merge(lo, n2, step)
            oddeven_merge(lo + r, n2, step)
            for i in range(lo + r, lo + n2 - r, step):
                pairs.append((i, i + r))
        else:
            pairs.append((lo, lo + r))

    def sort_range(lo, hi):
        if hi - lo >= 1:
            mid = lo + (hi - lo) // 2
            sort_range(lo, mid)
            sort_range(mid + 1, hi)
            oddeven_merge(lo, hi - lo + 1, 1)

    sort_range(0, n - 1)
    return pairs


_PAIRS32 = _oems_pairs(N_LISTS)


def _sort32_desc(Y):
    for i, j in _PAIRS32:
        a, b = Y[i], Y[j]
        Y[i], Y[j] = jnp.maximum(a, b), jnp.minimum(a, b)
    return Y


def _bitonic_merge32_desc(C):
    for d in (16, 8, 4, 2, 1):
        for i in range(N_LISTS):
            l = i + d
            if (i & d) == 0 and l < N_LISTS:
                a, b = C[i], C[l]
                C[i], C[l] = jnp.maximum(a, b), jnp.minimum(a, b)
    return C


STOP_L = 16


def _local_sorted_lists_t(x):
    _, c = x.shape
    L = c // N_LISTS
    Y = [x[:, L * j : L * (j + 1)] for j in range(N_LISTS)]
    Y = _sort32_desc(Y)
    while L > STOP_L:
        h = L // 2
        C = [
            jnp.maximum(Y[j][:, :h], Y[N_LISTS - 1 - j][:, h:])
            for j in range(N_LISTS)
        ]
        Y = _bitonic_merge32_desc(C)
        L = h
    mt = jnp.transpose(jnp.concatenate(Y, axis=1))
    lists = [
        [mt[STOP_L * j + l : STOP_L * j + l + 1, :] for j in range(N_LISTS)]
        for l in range(STOP_L)
    ]
    while len(lists) > 1:
        lists = [
            _merge2(lists[2 * i], lists[2 * i + 1])
            for i in range(len(lists) // 2)
        ]
    return lists[0]


def _row_list(A):
    return [A[j : j + 1, :] for j in range(N_LISTS)]


def _merge2(Al, Bl):
    C = [jnp.maximum(Al[j], Bl[N_LISTS - 1 - j]) for j in range(N_LISTS)]
    return _bitonic_merge32_desc(C)


def _body(
    x_hbm, out_ref, xb, loc_ref, comm1, comm2, cp_sem, s1, r1, s2, r2,
):
    my_x = lax.axis_index("x")
    my_y = lax.axis_index("y")
    my_z = lax.axis_index("z")
    r = _ring_index(my_x, my_y)

    def z_peer(dz):
        return (my_x, my_y, (my_z + dz) % Z)

    def xy_peer(d):
        px, py = _ring_coords((r + d) % P)
        return (px, py, my_z)

    peers = [z_peer(dz) for dz in (1, 2, 3)] + [xy_peer(d) for d in range(1, P)]

    barrier = pltpu.get_barrier_semaphore()
    for dev in peers:
        pl.semaphore_signal(
            barrier, inc=1, device_id=dev, device_id_type=pl.DeviceIdType.MESH
        )

    hb = RB // 2
    cps = []
    for h in range(2):
        cp = pltpu.make_async_copy(
            x_hbm.at[pl.ds(r * RB + h * hb, hb), :],
            xb.at[pl.ds(h * hb, hb), :],
            cp_sem.at[h],
        )
        cp.start()
        cps.append(cp)
    half_lists = []
    for h in range(2):
        cps[h].wait()
        half_lists.append(_local_sorted_lists_t(xb[h * hb : (h + 1) * hb, :]))
    mine_lists = [
        jnp.concatenate([half_lists[0][j], half_lists[1][j]], axis=1)
        for j in range(N_LISTS)
    ]
    loc_ref[...] = jnp.concatenate(mine_lists, axis=0)

    pl.semaphore_wait(barrier, len(peers))

    sends1 = []
    for dz in (1, 2, 3):
        rdma = pltpu.make_async_remote_copy(
            src_ref=loc_ref,
            dst_ref=comm1.at[dz - 1],
            send_sem=s1.at[dz - 1],
            recv_sem=r1.at[dz - 1],
            device_id=z_peer(dz),
            device_id_type=pl.DeviceIdType.MESH,
        )
        rdma.start()
        sends1.append(rdma)
    for rdma in sends1:
        rdma.wait_recv()

    m01 = _merge2(mine_lists, _row_list(comm1[0]))
    m23 = _merge2(_row_list(comm1[1]), _row_list(comm1[2]))
    merged_t = jnp.concatenate(_merge2(m01, m23), axis=0)

    comm2[pl.ds(r, 1), :, :] = merged_t[None]

    sends2 = []
    for d in range(1, P):
        rdma = pltpu.make_async_remote_copy(
            src_ref=comm2.at[r],
            dst_ref=comm2.at[r],
            send_sem=s2.at[d - 1],
            recv_sem=r2.at[d - 1],
            device_id=xy_peer(d),
            device_id_type=pl.DeviceIdType.MESH,
        )
        rdma.start()
        sends2.append(rdma)
    for rdma in sends2:
        rdma.wait_recv()

    t_all = jnp.concatenate([comm2[o] for o in range(P)], axis=1)
    out_ref[...] = jnp.transpose(t_all)

    for rdma in sends1:
        rdma.wait_send()
    for rdma in sends2:
        rdma.wait_send()

    @functools.partial(pl.run_scoped, sem=pltpu.SemaphoreType.REGULAR)
    def _(sem):
        for dev in peers:
            pl.semaphore_signal(
                sem, inc=1, device_id=dev, device_id_type=pl.DeviceIdType.MESH
            )
        pl.semaphore_wait(sem, len(peers))


def kernel(x):
    m, n = x.shape

    return pl.pallas_call(
        _body,
        out_shape=jax.ShapeDtypeStruct((m, K), jnp.float32),
        in_specs=[pl.BlockSpec(memory_space=pl.ANY)],
        out_specs=pl.BlockSpec(memory_space=pltpu.VMEM),
        scratch_shapes=[
            pltpu.VMEM((RB, n), jnp.float32),
            pltpu.VMEM((K, RB), jnp.float32),
            pltpu.VMEM((Z - 1, K, RB), jnp.float32),
            pltpu.VMEM((P, K, RB), jnp.float32),
            pltpu.SemaphoreType.DMA((2,)),
            pltpu.SemaphoreType.DMA((Z - 1,)),
            pltpu.SemaphoreType.DMA((Z - 1,)),
            pltpu.SemaphoreType.DMA((P - 1,)),
            pltpu.SemaphoreType.DMA((P - 1,)),
        ],
        compiler_params=pltpu.CompilerParams(collective_id=0),
    )(x)


# device time: 21191 ns/iter; 5.7548x vs baseline; 1.0068x over previous
import functools

import jax
import jax.numpy as jnp
from jax import lax
from jax.experimental import pallas as pl
from jax.experimental.pallas import tpu as pltpu

N_LISTS = 32
K = 32
Z = 4
P = 8
RB = 128


def _ring_index(x_idx, y_idx):
    return jnp.where(x_idx == 0, y_idx, 2 * Z - 1 - y_idx)


def _ring_coords(p):
    px = p // Z
    py = jnp.where(px == 0, p, 2 * Z - 1 - p)
    return px, py


def _oems_pairs(n):
    pairs = []

    def oddeven_merge(lo, n2, r):
        step = r * 2
        if step < n2:
            oddeven_merge(lo, n2, step)
            oddeven_merge(lo + r, n2, step)
            for i in range(lo + r, lo + n2 - r, step):
                pairs.append((i, i + r))
        else:
            pairs.append((lo, lo + r))

    def sort_range(lo, hi):
        if hi - lo >= 1:
            mid = lo + (hi - lo) // 2
            sort_range(lo, mid)
            sort_range(mid + 1, hi)
            oddeven_merge(lo, hi - lo + 1, 1)

    sort_range(0, n - 1)
    return pairs


_PAIRS32 = _oems_pairs(N_LISTS)


def _sort32_desc(Y):
    for i, j in _PAIRS32:
        a, b = Y[i], Y[j]
        Y[i], Y[j] = jnp.maximum(a, b), jnp.minimum(a, b)
    return Y


def _bitonic_merge32_desc(C):
    for d in (16, 8, 4, 2, 1):
        for i in range(N_LISTS):
            l = i + d
            if (i & d) == 0 and l < N_LISTS:
                a, b = C[i], C[l]
                C[i], C[l] = jnp.maximum(a, b), jnp.minimum(a, b)
    return C


STOP_L = 16


def _local_sorted_lists_t(x):
    _, c = x.shape
    L = c // N_LISTS
    Y = [x[:, L * j : L * (j + 1)] for j in range(N_LISTS)]
    Y = _sort32_desc(Y)
    while L > STOP_L:
        h = L // 2
        C = [
            jnp.maximum(Y[j][:, :h], Y[N_LISTS - 1 - j][:, h:])
            for j in range(N_LISTS)
        ]
        Y = _bitonic_merge32_desc(C)
        L = h
    mt = jnp.transpose(jnp.concatenate(Y, axis=1))
    S = [mt[STOP_L * j : STOP_L * (j + 1), :] for j in range(N_LISTS)]
    w = STOP_L
    while w > 1:
        h = w // 2
        C = [
            jnp.maximum(S[j][:h, :], S[N_LISTS - 1 - j][h:, :])
            for j in range(N_LISTS)
        ]
        S = _bitonic_merge32_desc(C)
        w = h
    return S


def _row_list(A):
    return [A[j : j + 1, :] for j in range(N_LISTS)]


def _merge2(Al, Bl):
    C = [jnp.maximum(Al[j], Bl[N_LISTS - 1 - j]) for j in range(N_LISTS)]
    return _bitonic_merge32_desc(C)


def _body(
    x_hbm, out_ref, xb, loc_ref, comm1, comm2, cp_sem, s1, r1, s2, r2,
):
    my_x = lax.axis_index("x")
    my_y = lax.axis_index("y")
    my_z = lax.axis_index("z")
    r = _ring_index(my_x, my_y)

    def z_peer(dz):
        return (my_x, my_y, (my_z + dz) % Z)

    def xy_peer(d):
        px, py = _ring_coords((r + d) % P)
        return (px, py, my_z)

    peers = [z_peer(dz) for dz in (1, 2, 3)] + [xy_peer(d) for d in range(1, P)]

    barrier = pltpu.get_barrier_semaphore()
    for dev in peers:
        pl.semaphore_signal(
            barrier, inc=1, device_id=dev, device_id_type=pl.DeviceIdType.MESH
        )

    hb = RB // 2
    cps = []
    for h in range(2):
        cp = pltpu.make_async_copy(
            x_hbm.at[pl.ds(r * RB + h * hb, hb), :],
            xb.at[pl.ds(h * hb, hb), :],
            cp_sem.at[h],
        )
        cp.start()
        cps.append(cp)
    half_lists = []
    for h in range(2):
        cps[h].wait()
        half_lists.append(_local_sorted_lists_t(xb[h * hb : (h + 1) * hb, :]))
    mine_lists = [
        jnp.concatenate([half_lists[0][j], half_lists[1][j]], axis=1)
        for j in range(N_LISTS)
    ]
    loc_ref[...] = jnp.concatenate(mine_lists, axis=0)

    pl.semaphore_wait(barrier, len(peers))

    sends1 = []
    for dz in (1, 2, 3):
        rdma = pltpu.make_async_remote_copy(
            src_ref=loc_ref,
            dst_ref=comm1.at[dz - 1],
            send_sem=s1.at[dz - 1],
            recv_sem=r1.at[dz - 1],
            device_id=z_peer(dz),
            device_id_type=pl.DeviceIdType.MESH,
        )
        rdma.start()
        sends1.append(rdma)
    sends1[0].wait_recv()
    m01 = _merge2(mine_lists, _row_list(comm1[0]))
    sends1[1].wait_recv()
    sends1[2].wait_recv()
    m23 = _merge2(_row_list(comm1[1]), _row_list(comm1[2]))
    merged_t = jnp.concatenate(_merge2(m01, m23), axis=0)

    comm2[pl.ds(r, 1), :, :] = merged_t[None]

    sends2 = []
    for d in range(1, P):
        rdma = pltpu.make_async_remote_copy(
            src_ref=comm2.at[r],
            dst_ref=comm2.at[r],
            send_sem=s2.at[d - 1],
            recv_sem=r2.at[d - 1],
            device_id=xy_peer(d),
            device_id_type=pl.DeviceIdType.MESH,
        )
        rdma.start()
        sends2.append(rdma)
    for rdma in sends2:
        rdma.wait_recv()

    t_all = jnp.concatenate([comm2[o] for o in range(P)], axis=1)
    out_ref[...] = jnp.transpose(t_all)

    for rdma in sends1:
        rdma.wait_send()
    for rdma in sends2:
        rdma.wait_send()

    @functools.partial(pl.run_scoped, sem=pltpu.SemaphoreType.REGULAR)
    def _(sem):
        for dev in peers:
            pl.semaphore_signal(
                sem, inc=1, device_id=dev, device_id_type=pl.DeviceIdType.MESH
            )
        pl.semaphore_wait(sem, len(peers))


def kernel(x):
    m, n = x.shape

    return pl.pallas_call(
        _body,
        out_shape=jax.ShapeDtypeStruct((m, K), jnp.float32),
        in_specs=[pl.BlockSpec(memory_space=pl.ANY)],
        out_specs=pl.BlockSpec(memory_space=pltpu.VMEM),
        scratch_shapes=[
            pltpu.VMEM((RB, n), jnp.float32),
            pltpu.VMEM((K, RB), jnp.float32),
            pltpu.VMEM((Z - 1, K, RB), jnp.float32),
            pltpu.VMEM((P, K, RB), jnp.float32),
            pltpu.SemaphoreType.DMA((2,)),
            pltpu.SemaphoreType.DMA((Z - 1,)),
            pltpu.SemaphoreType.DMA((Z - 1,)),
            pltpu.SemaphoreType.DMA((P - 1,)),
            pltpu.SemaphoreType.DMA((P - 1,)),
        ],
        compiler_params=pltpu.CompilerParams(collective_id=0),
    )(x)
